# baseline (speedup 1.0000x reference)
import functools

import jax
import jax.numpy as jnp
from jax import lax
from jax.experimental import pallas as pl
from jax.experimental.pallas import tpu as pltpu
from jax.experimental.pallas import tpu_sc as plsc

N_THETA = 181
N_PHI = 360
PHI_PERIOD = 360.0
GRID_N = N_THETA * N_PHI

NC = 2
NS = 16
L = 16
NW = NC * NS


def _build_interp(n_total: int, chunk: int, unroll: int):
  assert n_total % (NW * chunk) == 0
  per_w = n_total // NW
  n_chunks = per_w // chunk
  assert n_chunks % 2 == 0 and chunk % (unroll * L) == 0

  mesh = plsc.VectorSubcoreMesh(
      core_axis_name="c", subcore_axis_name="s", num_cores=NC, num_subcores=NS
  )

  def body(th_hbm, ph_hbm, grid_hbm, par_hbm, out_hbm,
           grid_v, par_v, th_v, ph_v, out_v,
           sp_grid, sp_in, sp_out,
           in_sem0, in_sem1, out_sem0, out_sem1, g_sem):
    cid = lax.axis_index("c")
    sid = lax.axis_index("s")
    wid = sid * NC + cid
    base = wid * per_w
    in_sems = (in_sem0, in_sem1)
    out_sems = (out_sem0, out_sem1)

    def sp_in_slice(b, which):
      return sp_in.at[pl.ds(((sid * 2 + b) * 2 + which) * chunk, chunk)]

    def sp_out_slice(b):
      return sp_out.at[pl.ds((sid * 2 + b) * chunk, chunk)]

    def fire_in(ci, b):
      off = base + ci * chunk
      pltpu.async_copy(th_hbm.at[pl.ds(off, chunk)], sp_in_slice(b, 0),
                       in_sems[b])
      pltpu.async_copy(ph_hbm.at[pl.ds(off, chunk)], sp_in_slice(b, 1),
                       in_sems[b])

    def wait_in(b):
      pltpu.make_async_copy(th_hbm.at[pl.ds(0, chunk)], sp_in_slice(b, 0),
                            in_sems[b]).wait()
      pltpu.make_async_copy(ph_hbm.at[pl.ds(0, chunk)], sp_in_slice(b, 1),
                            in_sems[b]).wait()

    def fire_out(ci, b):
      off = base + ci * chunk
      pltpu.async_copy(sp_out_slice(b), out_hbm.at[pl.ds(off, chunk)],
                       out_sems[b])

    def wait_out(b):
      pltpu.make_async_copy(sp_out_slice(b), out_hbm.at[pl.ds(0, chunk)],
                            out_sems[b]).wait()

    fire_in(0, 0)

    @pl.when(sid == 0)
    def _():
      pltpu.async_copy(grid_hbm, sp_grid, g_sem).wait()

    plsc.subcore_barrier()
    pltpu.sync_copy(sp_grid, grid_v)
    pltpu.sync_copy(par_hbm, par_v)
    tc0 = par_v[pl.ds(0, L)]
    tcL = par_v[pl.ds(L, L)]
    inv_dt = par_v[pl.ds(2 * L, L)]
    pc0 = par_v[pl.ds(3 * L, L)]
    pcL = par_v[pl.ds(4 * L, L)]
    inv_dp = par_v[pl.ds(5 * L, L)]

    def compute():
      @plsc.parallel_loop(0, chunk, step=L, unroll=unroll)
      def _vec(i):
        s = pl.ds(i, L)
        th = th_v[s]
        ph = ph_v[s]
        thc = jnp.minimum(jnp.maximum(th, tc0), tcL)
        ut = (thc - tc0) * inv_dt
        it0 = jnp.minimum(ut.astype(jnp.int32), N_THETA - 2)
        tt = ut - it0.astype(jnp.float32)
        q = (ph - pc0) * (1.0 / PHI_PERIOD) + 4.0
        k = q.astype(jnp.int32).astype(jnp.float32) - 4.0
        wr = ph - k * PHI_PERIOD
        phc = jnp.minimum(jnp.maximum(wr, pc0), pcL)
        up = (phc - pc0) * inv_dp
        ip0 = jnp.minimum(up.astype(jnp.int32), N_PHI - 2)
        tp = up - ip0.astype(jnp.float32)
        f00 = it0 * N_PHI + ip0
        a = plsc.load_gather(grid_v, [f00])
        bb = plsc.load_gather(grid_v, [f00 + 1])
        c = plsc.load_gather(grid_v, [f00 + N_PHI])
        d = plsc.load_gather(grid_v, [f00 + (N_PHI + 1)])
        e0 = a + tp * (bb - a)
        e1 = c + tp * (d - c)
        out_v[s] = e0 + tt * (e1 - e0)

    def group_fn(g, carry):
      for b in range(2):
        ci = 2 * g + b
        pl.when(ci + 1 < n_chunks)(lambda: fire_in(ci + 1, 1 - b))
        wait_in(b)
        pltpu.sync_copy(sp_in_slice(b, 0), th_v)
        pltpu.sync_copy(sp_in_slice(b, 1), ph_v)
        compute()
        pl.when(ci >= 2)(lambda: wait_out(b))
        pltpu.sync_copy(out_v, sp_out_slice(b))
        fire_out(ci, b)
      return carry

    lax.fori_loop(0, n_chunks // 2, group_fn, 0)
    wait_out(0)
    wait_out(1)

  return pl.kernel(
      body,
      out_type=jax.ShapeDtypeStruct((n_total,), jnp.float32),
      mesh=mesh,
      compiler_params=pltpu.CompilerParams(needs_layout_passes=False),
      scratch_types=[
          pltpu.VMEM((GRID_N,), jnp.float32),
          pltpu.VMEM((6 * L,), jnp.float32),
          pltpu.VMEM((chunk,), jnp.float32),
          pltpu.VMEM((chunk,), jnp.float32),
          pltpu.VMEM((chunk,), jnp.float32),
          pltpu.VMEM_SHARED((GRID_N,), jnp.float32),
          pltpu.VMEM_SHARED((NS * 2 * 2 * chunk,), jnp.float32),
          pltpu.VMEM_SHARED((NS * 2 * chunk,), jnp.float32),
          pltpu.SemaphoreType.DMA,
          pltpu.SemaphoreType.DMA,
          pltpu.SemaphoreType.DMA,
          pltpu.SemaphoreType.DMA,
          pltpu.SemaphoreType.DMA,
      ],
  )


@jax.jit
def kernel(theta_deg, phi_deg, energy_grid, theta_centers, phi_centers):
  orig_shape = theta_deg.shape
  th = theta_deg.reshape(-1)
  ph = phi_deg.reshape(-1)
  grid = energy_grid.reshape(-1)
  tc, pc = theta_centers, phi_centers
  scalars = (tc[0], tc[-1], 1.0 / (tc[1] - tc[0]),
             pc[0], pc[-1], 1.0 / (pc[1] - pc[0]))
  params = jnp.concatenate(
      [jnp.full((L,), s, dtype=jnp.float32) for s in scalars])
  interp = _build_interp(th.shape[0], 6400, 8)
  out = interp(th, ph, grid, params)
  return out.reshape(orig_shape)
